# Initial kernel scaffold; baseline (speedup 1.0000x reference)
import jax
import jax.numpy as jnp
from jax.experimental import pallas as pl


def kernel(x, edge_index, W0, al0, ar0, b0, W2, al2, ar2, b2):
    raise NotImplementedError("write your pallas kernel here")


# speedup vs baseline: 9.9219x; 9.9219x over previous
import functools

import jax
import jax.numpy as jnp
from jax import lax
from jax.experimental import pallas as pl
from jax.experimental.pallas import tpu as pltpu
from jax.experimental.pallas import tpu_sc as plsc

N = 10000
E = 320000
IN_DIM = 128
HID = 128
H0 = 4
NC = 40

BN = 1000
D1 = HID
DP1 = HID + 16
D2 = 48
DP2 = 48
NSUB = 16
NROW = N // NSUB
EB = 64
CS = 1024

CHUNK1 = E // NSUB
CHUNK2 = E // (2 * NSUB)
NCH1 = (CHUNK1 + CS - 1) // CS
NCH2 = (CHUNK2 + CS - 1) // CS
E_PAD = NSUB * NCH1 * CS


def _proj1_body(x_ref, w_ref, al_ref, ar_ref, feat_ref, el_ref, er_ref):
    feat = jnp.dot(x_ref[...], w_ref[...], preferred_element_type=jnp.float32)
    feat_ref[0] = feat
    el_ref[0] = jnp.sum(feat * al_ref[0], axis=-1)[None, :]
    er_ref[0] = jnp.sum(feat * ar_ref[0], axis=-1)[None, :]


def _tc_proj1(x, W0, al0, ar0):
    grid = (H0,)
    return pl.pallas_call(
        _proj1_body,
        grid=grid,
        in_specs=[
            pl.BlockSpec((N, IN_DIM), lambda h: (0, 0)),
            pl.BlockSpec((IN_DIM, D1), lambda h: (0, h)),
            pl.BlockSpec((1, 1, D1), lambda h: (h, 0, 0)),
            pl.BlockSpec((1, 1, D1), lambda h: (h, 0, 0)),
        ],
        out_specs=[
            pl.BlockSpec((1, N, D1), lambda h: (h, 0, 0)),
            pl.BlockSpec((1, 1, N), lambda h: (h, 0, 0)),
            pl.BlockSpec((1, 1, N), lambda h: (h, 0, 0)),
        ],
        out_shape=[
            jax.ShapeDtypeStruct((H0, N, D1), jnp.float32),
            jax.ShapeDtypeStruct((H0, 1, N), jnp.float32),
            jax.ShapeDtypeStruct((H0, 1, N), jnp.float32),
        ],
    )(x, W0.reshape(IN_DIM, H0 * D1), al0.reshape(H0, 1, D1),
      ar0.reshape(H0, 1, D1))


def _epi1_body(num_ref, b_ref, w2_ref, al_ref, ar_ref,
               feat_ref, el_ref, er_ref):
    parts = []
    for h in range(H0):
        blk = num_ref[h]
        den = blk[:, D1:D1 + 1]
        parts.append(blk[:, :D1] / (den + 1e-9))
    hfeat = jnp.concatenate(parts, axis=-1) + b_ref[...]
    hfeat = jnp.maximum(hfeat, 0.0)
    f2 = jnp.dot(hfeat, w2_ref[...], preferred_element_type=jnp.float32)
    feat_ref[...] = jnp.concatenate(
        [f2, jnp.zeros((BN, D2 - NC), jnp.float32)], axis=-1)
    el_ref[...] = jnp.sum(f2 * al_ref[...], axis=-1, keepdims=True)
    er_ref[...] = jnp.sum(f2 * ar_ref[...], axis=-1, keepdims=True)


def _tc_epi1(num1, b0, W2, al2, ar2):
    grid = (N // BN,)
    return pl.pallas_call(
        _epi1_body,
        grid=grid,
        in_specs=[
            pl.BlockSpec((H0, BN, DP1), lambda i: (0, i, 0)),
            pl.BlockSpec((1, H0 * HID), lambda i: (0, 0)),
            pl.BlockSpec((H0 * HID, NC), lambda i: (0, 0)),
            pl.BlockSpec((1, NC), lambda i: (0, 0)),
            pl.BlockSpec((1, NC), lambda i: (0, 0)),
        ],
        out_specs=[
            pl.BlockSpec((BN, D2), lambda i: (i, 0)),
            pl.BlockSpec((BN, 1), lambda i: (i, 0)),
            pl.BlockSpec((BN, 1), lambda i: (i, 0)),
        ],
        out_shape=[
            jax.ShapeDtypeStruct((N, D2), jnp.float32),
            jax.ShapeDtypeStruct((N, 1), jnp.float32),
            jax.ShapeDtypeStruct((N, 1), jnp.float32),
        ],
    )(num1, b0, W2, al2, ar2)


def _epi2_body(num_ref, b_ref, out_ref):
    s = num_ref[0] + num_ref[1]
    den = s[:, NC:NC + 1]
    out_ref[...] = s[:, :NC] / (den + 1e-9) + b_ref[...]


def _tc_epi2(num2, b2):
    grid = (N // BN,)
    return pl.pallas_call(
        _epi2_body,
        grid=grid,
        in_specs=[
            pl.BlockSpec((2, BN, DP2), lambda i: (0, i, 0)),
            pl.BlockSpec((1, NC), lambda i: (0, 0)),
        ],
        out_specs=pl.BlockSpec((BN, NC), lambda i: (i, 0)),
        out_shape=jax.ShapeDtypeStruct((N, NC), jnp.float32),
    )(num2, b2)


def _edge_batches(nchunks, nbatch_per_chunk, chunk_real, tile_base,
                  d_width, dp_groups, w_col, row_off,
                  src_hbm, dst_hbm, src_c, dst_c, el_v, er_v,
                  feat_hbm, num_sh, gidx, sidx, wbuf, rows, scaled, sem):
    lane = lax.iota(jnp.int32, 16)

    @pl.loop(0, nchunks)
    def _chunk(r):
        pltpu.sync_copy(src_hbm.at[pl.ds(tile_base + r * CS, CS)], src_c)
        pltpu.sync_copy(dst_hbm.at[pl.ds(tile_base + r * CS, CS)], dst_c)

        @pl.loop(0, nbatch_per_chunk)
        def _batch(b):
            off = b * EB
            goff = r * CS + b * EB
            for g in range(EB // 16):
                eoff = off + g * 16
                valid = (goff + g * 16 + lane) < chunk_real
                s16 = jnp.where(valid, src_c[pl.ds(eoff, 16)], 0)
                d16 = jnp.where(valid, dst_c[pl.ds(eoff, 16)], 0)
                e = plsc.load_gather(el_v, [s16]) + plsc.load_gather(er_v, [d16])
                e = jnp.where(e < 0, e * 0.2, e)
                w = jnp.where(valid, jnp.exp(e), 0.0)
                gidx[pl.ds(g * 16, 16)] = s16 + row_off
                sidx[pl.ds(g * 16, 16)] = d16
                wbuf[pl.ds(g * 16, 16)] = w
            pltpu.async_copy(feat_hbm.at[gidx], rows, sem).wait()

            @pl.loop(0, EB)
            def _scale(j):
                wv = plsc.load_gather(wbuf, [lane * 0 + j])
                for k in range(dp_groups):
                    base = k * 16
                    if base < d_width:
                        val = rows[j, pl.ds(base, 16)] * wv
                    else:
                        val = jnp.zeros((16,), jnp.float32)
                    if base <= w_col < base + 16:
                        val = jnp.where(lane == (w_col - base), wv, val)
                    scaled[j, pl.ds(base, 16)] = val

            pltpu.sync_copy(scaled, num_sh.at[sidx], add=True)


def _zero_accum_slice(s, scaled, num_sh, dp):
    @pl.loop(0, EB)
    def _z(j):
        for k in range(dp // 16):
            scaled[j, pl.ds(k * 16, 16)] = jnp.zeros((16,), jnp.float32)

    done = 0
    while done < NROW:
        step = min(EB, NROW - done)
        pltpu.sync_copy(scaled.at[pl.ds(0, step)],
                        num_sh.at[pl.ds(s * NROW + done, step)])
        done += step


def _sc_l1_body(feat_hbm, el_hbm, er_hbm, src_hbm, dst_hbm, out_hbm,
                src_c, dst_c, el_v, er_v, gidx, sidx, wbuf, rows, scaled,
                num_sh, sem):
    c = lax.axis_index("c")
    s = lax.axis_index("s")
    tile_base = s * CHUNK1
    for hp in range(2):
        h = c * 2 + hp
        pltpu.sync_copy(el_hbm.at[pl.ds(h * N, N)], el_v)
        pltpu.sync_copy(er_hbm.at[pl.ds(h * N, N)], er_v)
        _zero_accum_slice(s, scaled, num_sh, DP1)
        plsc.subcore_barrier()
        _edge_batches(NCH1, CS // EB, CHUNK1, tile_base,
                      D1, DP1 // 16, D1, h * N,
                      src_hbm, dst_hbm, src_c, dst_c, el_v, er_v,
                      feat_hbm, num_sh, gidx, sidx, wbuf, rows, scaled, sem)
        plsc.subcore_barrier()
        pltpu.sync_copy(num_sh.at[pl.ds(s * NROW, NROW)],
                        out_hbm.at[pl.ds(h * N + s * NROW, NROW)])
        plsc.subcore_barrier()


def _sc_layer1(featT, el, er, src, dst):
    mesh = plsc.VectorSubcoreMesh(core_axis_name="c", subcore_axis_name="s")
    return pl.kernel(
        _sc_l1_body,
        out_type=jax.ShapeDtypeStruct((H0 * N, DP1), jnp.float32),
        mesh=mesh,
        compiler_params=pltpu.CompilerParams(
            use_tc_tiling_on_sc=False, needs_layout_passes=False),
        scratch_types=[
            pltpu.VMEM((CS,), jnp.int32),
            pltpu.VMEM((CS,), jnp.int32),
            pltpu.VMEM((N,), jnp.float32),
            pltpu.VMEM((N,), jnp.float32),
            pltpu.VMEM((EB,), jnp.int32),
            pltpu.VMEM((EB,), jnp.int32),
            pltpu.VMEM((EB,), jnp.float32),
            pltpu.VMEM((EB, D1), jnp.float32),
            pltpu.VMEM((EB, DP1), jnp.float32),
            pltpu.VMEM_SHARED((N, DP1), jnp.float32),
            pltpu.SemaphoreType.DMA,
        ],
    )(featT, el, er, src, dst)


def _sc_l2_body(feat_hbm, el_hbm, er_hbm, src_hbm, dst_hbm, out_hbm,
                src_c, dst_c, el_v, er_v, gidx, sidx, wbuf, rows, scaled,
                num_sh, sem):
    c = lax.axis_index("c")
    s = lax.axis_index("s")
    tile_base = (c * NSUB + s) * CHUNK2
    pltpu.sync_copy(el_hbm, el_v)
    pltpu.sync_copy(er_hbm, er_v)
    _zero_accum_slice(s, scaled, num_sh, DP2)
    plsc.subcore_barrier()
    _edge_batches(NCH2, CS // EB, CHUNK2, tile_base,
                  D2, DP2 // 16, NC, 0,
                  src_hbm, dst_hbm, src_c, dst_c, el_v, er_v,
                  feat_hbm, num_sh, gidx, sidx, wbuf, rows, scaled, sem)
    plsc.subcore_barrier()
    pltpu.sync_copy(num_sh.at[pl.ds(s * NROW, NROW)],
                    out_hbm.at[pl.ds(c * N + s * NROW, NROW)])


def _sc_layer2(feat2, el2, er2, src, dst):
    mesh = plsc.VectorSubcoreMesh(core_axis_name="c", subcore_axis_name="s")
    return pl.kernel(
        _sc_l2_body,
        out_type=jax.ShapeDtypeStruct((2 * N, DP2), jnp.float32),
        mesh=mesh,
        compiler_params=pltpu.CompilerParams(
            use_tc_tiling_on_sc=False, needs_layout_passes=False),
        scratch_types=[
            pltpu.VMEM((CS,), jnp.int32),
            pltpu.VMEM((CS,), jnp.int32),
            pltpu.VMEM((N,), jnp.float32),
            pltpu.VMEM((N,), jnp.float32),
            pltpu.VMEM((EB,), jnp.int32),
            pltpu.VMEM((EB,), jnp.int32),
            pltpu.VMEM((EB,), jnp.float32),
            pltpu.VMEM((EB, D2), jnp.float32),
            pltpu.VMEM((EB, DP2), jnp.float32),
            pltpu.VMEM_SHARED((N, DP2), jnp.float32),
            pltpu.SemaphoreType.DMA,
        ],
    )(feat2, el2, er2, src, dst)


@jax.jit
def _run(x, edge_index, W0, al0, ar0, b0, W2, al2, ar2, b2):
    src = jnp.pad(edge_index[0], (0, E_PAD - E))
    dst = jnp.pad(edge_index[1], (0, E_PAD - E))
    featT, el1, er1 = _tc_proj1(x, W0, al0, ar0)
    num1 = _sc_layer1(featT.reshape(H0 * N, D1),
                      el1.reshape(H0 * N), er1.reshape(H0 * N), src, dst)
    feat2, el2, er2 = _tc_epi1(num1.reshape(H0, N, DP1),
                               b0.reshape(1, H0 * HID), W2, al2, ar2)
    num2 = _sc_layer2(feat2, el2.reshape(N), er2.reshape(N), src, dst)
    return _tc_epi2(num2.reshape(2, N, DP2), b2.reshape(1, NC))


def kernel(x, edge_index, W0, al0, ar0, b0, W2, al2, ar2, b2):
    return _run(x, edge_index, W0, al0, ar0, b0, W2, al2, ar2, b2)
